# Initial kernel scaffold; baseline (speedup 1.0000x reference)
import jax
import jax.numpy as jnp
from jax.experimental import pallas as pl


def kernel(x, edge_index, edge_weight, W1, a_src1, a_dst1, b1, W2, a_src2, a_dst2, b2):
    raise NotImplementedError("write your pallas kernel here")


# speedup vs baseline: 27.5518x; 27.5518x over previous
import functools

import jax
import jax.numpy as jnp
from jax import lax
from jax.experimental import pallas as pl
from jax.experimental.pallas import tpu as pltpu
from jax.experimental.pallas import tpu_sc as plsc

N = 10000
E = 160000
F_IN = 256
HEADS = 8
HID = 8
NCLS = 16
NEG = 0.2

CH = 128
NT = 32
E1 = E + N
K1 = -(-E1 // (NT * CH))
E_PAD = NT * CH * K1
NROW = 10112
RPT = NROW // 16
W = 128

_f32 = jnp.float32
_i32 = jnp.int32


def _lrelu(a):
    return jnp.where(a > 0, a, NEG * a)


def _take(x, idx):
    return x.at[idx].get(mode="promise_in_bounds")


def _tc1_body(x_ref, w1_ref, as_ref, ad_ref, htab_ref):
    h = jnp.dot(x_ref[...], w1_ref[...], preferred_element_type=_f32)
    es = jnp.dot(h, as_ref[...], preferred_element_type=_f32)
    ed = jnp.dot(h, ad_ref[...], preferred_element_type=_f32)
    e = jnp.concatenate([es, ed], axis=1)
    z48 = jnp.zeros_like(h[:, :48])
    htab_ref[...] = jnp.concatenate([h, e, z48], axis=1)


def _tc2_body(acc_ref, w2_ref, r_ref, b1_ref, a2s_ref, a2d_ref,
              hid_ref, t2_ref):
    s = acc_ref[0] + acc_ref[1]
    msg = s[:, :64]
    den = s[:, 64:72]
    deni = 1.0 / (den + 1e-16)
    rep = jnp.dot(deni, r_ref[...], preferred_element_type=_f32)
    v = msg * rep + b1_ref[...]
    hid = jnp.where(v > 0, v, jnp.exp(v) - 1.0)
    hid_ref[...] = hid
    h2 = jnp.dot(hid, w2_ref[...], preferred_element_type=_f32)
    e2s = jnp.dot(h2, a2s_ref[...], preferred_element_type=_f32)
    e2d = jnp.dot(h2, a2d_ref[...], preferred_element_type=_f32)
    z110 = jnp.zeros((h2.shape[0], 110), _f32)
    t2_ref[...] = jnp.concatenate([h2, e2s, e2d, z110], axis=1)


def _tc3_body(acc_ref, b2_ref, out_ref):
    s = acc_ref[0] + acc_ref[1]
    den = s[:, 16:17]
    out_ref[...] = s * (1.0 / (den + 1e-16)) + b2_ref[...]


def _sc1_body(htab_hbm, c1_hbm, src_hbm, dst_hbm, ew_hbm, z_hbm,
              out_hbm,
              acc_sh, srcb, dstb, ewb, hbuf, edb, valb, c1b, sem):
    cidx = lax.axis_index("c")
    sidx = lax.axis_index("s")
    wid = sidx * 2 + cidx

    pltpu.sync_copy(c1_hbm, c1b)
    pltpu.sync_copy(z_hbm, acc_sh.at[pl.ds(sidx * RPT, RPT)])
    pltpu.sync_copy(z_hbm.at[pl.ds(0, CH)], valb)
    plsc.subcore_barrier()

    base = wid * (K1 * CH)
    iota = lax.iota(_i32, 16)
    c1vec = c1b[...]
    shf = (iota & 7) + 8
    widx = [(iota >> 3) + 2 * v for v in range(4)]

    def chunk(k, carry):
        off = base + k * CH
        pltpu.sync_copy(src_hbm.at[pl.ds(off, CH)], srcb)
        pltpu.sync_copy(dst_hbm.at[pl.ds(off, CH)], dstb)
        pltpu.sync_copy(ew_hbm.at[pl.ds(off, CH)], ewb)
        d1 = pltpu.async_copy(htab_hbm.at[srcb], hbuf, sem)
        d2 = pltpu.async_copy(htab_hbm.at[dstb], edb, sem)
        d1.wait()
        d2.wait()
        def group(g, c0):
            ew16 = ewb[pl.ds(g * 16, 16)]

            def edge(j, c1):
                e = g * 16 + j
                esr = hbuf[e, pl.ds(64, 16)]
                edr = edb[e, pl.ds(64, 16)]
                s = esr + _take(edr, shf)
                av = _lrelu(s) - c1vec
                ex = jnp.exp(av)
                wj = _take(ew16, jnp.zeros((16,), _i32) + j)
                exw = ex * wj
                for v in range(4):
                    hv = hbuf[e, pl.ds(v * 16, 16)]
                    valb[e, pl.ds(v * 16, 16)] = hv * _take(exw, widx[v])
                valb[e, pl.ds(64, 16)] = ex
                return c1

            lax.fori_loop(0, 16, edge, 0)
            return c0

        lax.fori_loop(0, CH // 16, group, 0)
        pltpu.sync_copy(valb, acc_sh.at[dstb], add=True)
        return carry

    lax.fori_loop(0, K1, chunk, 0)
    plsc.subcore_barrier()
    pltpu.sync_copy(acc_sh.at[pl.ds(sidx * RPT, RPT)],
                    out_hbm.at[cidx, pl.ds(sidx * RPT, RPT)])


def _sc2_body(t2_hbm, c2_hbm, src_hbm, dst_hbm, ew_hbm, z_hbm,
              out_hbm,
              acc_sh, srcb, dstb, ewb, abuf, bbuf, valb, c2b, sem):
    cidx = lax.axis_index("c")
    sidx = lax.axis_index("s")
    wid = sidx * 2 + cidx

    pltpu.sync_copy(c2_hbm, c2b)
    pltpu.sync_copy(z_hbm, acc_sh.at[pl.ds(sidx * RPT, RPT)])
    pltpu.sync_copy(z_hbm.at[pl.ds(0, CH)], valb)
    plsc.subcore_barrier()

    base = wid * (K1 * CH)
    iota = lax.iota(_i32, 16)
    zidx = iota & 0
    onei = zidx + 1
    c2vec = c2b[...]

    def chunk(k, carry):
        off = base + k * CH
        pltpu.sync_copy(src_hbm.at[pl.ds(off, CH)], srcb)
        pltpu.sync_copy(dst_hbm.at[pl.ds(off, CH)], dstb)
        pltpu.sync_copy(ew_hbm.at[pl.ds(off, CH)], ewb)
        d1 = pltpu.async_copy(t2_hbm.at[srcb], abuf, sem)
        d2 = pltpu.async_copy(t2_hbm.at[dstb], bbuf, sem)
        d1.wait()
        d2.wait()
        def group(g, c0):
            ew16 = ewb[pl.ds(g * 16, 16)]

            def edge(j, c1):
                e = g * 16 + j
                m = abuf[e, pl.ds(0, 16)]
                sv = abuf[e, pl.ds(16, 16)]
                edr = bbuf[e, pl.ds(16, 16)]
                s = sv + _take(edr, onei)
                av = _lrelu(s) - c2vec
                ex = jnp.exp(av)
                wj = _take(ew16, jnp.zeros((16,), _i32) + j)
                exw = ex * wj
                valb[e, pl.ds(0, 16)] = m * _take(exw, zidx)
                valb[e, pl.ds(16, 16)] = ex
                return c1

            lax.fori_loop(0, 16, edge, 0)
            return c0

        lax.fori_loop(0, CH // 16, group, 0)
        pltpu.sync_copy(valb, acc_sh.at[dstb], add=True)
        return carry

    lax.fori_loop(0, K1, chunk, 0)
    plsc.subcore_barrier()
    pltpu.sync_copy(acc_sh.at[pl.ds(sidx * RPT, RPT)],
                    out_hbm.at[cidx, pl.ds(sidx * RPT, RPT)])


def kernel(x, edge_index, edge_weight, W1, a_src1, a_dst1, b1,
           W2, a_src2, a_dst2, b2):
    loop = jnp.arange(N, dtype=edge_index.dtype)
    pad = E_PAD - E1
    src = jnp.concatenate([edge_index[0], loop,
                           jnp.zeros((pad,), edge_index.dtype)])
    dst = jnp.concatenate([edge_index[1], loop,
                           jnp.full((pad,), N, edge_index.dtype)])
    ew = jnp.concatenate([edge_weight, jnp.ones((N,), _f32),
                          jnp.zeros((pad,), _f32)])

    eye8 = jnp.eye(HEADS, dtype=_f32)
    A_s = (a_src1[:, :, None] * eye8[:, None, :]).reshape(HEADS * HID, HEADS)
    A_d = (a_dst1[:, :, None] * eye8[:, None, :]).reshape(HEADS * HID, HEADS)
    R = jnp.repeat(eye8, HID, axis=1)

    BLK = 1000
    G = N // BLK

    htab = pl.pallas_call(
        _tc1_body,
        grid=(G,),
        in_specs=[pl.BlockSpec((BLK, F_IN), lambda i: (i, 0)),
                  pl.BlockSpec((F_IN, 64), lambda i: (0, 0)),
                  pl.BlockSpec((64, 8), lambda i: (0, 0)),
                  pl.BlockSpec((64, 8), lambda i: (0, 0))],
        out_specs=pl.BlockSpec((BLK, W), lambda i: (i, 0)),
        out_shape=jax.ShapeDtypeStruct((N, W), _f32),
    )(x, W1, A_s, A_d)

    mxv = jnp.max(htab[:, 64:80], axis=0)
    c1 = _lrelu(mxv[:8] + mxv[8:])
    c1v = jnp.concatenate([c1, jnp.zeros((8,), _f32)])
    htab_p = jnp.zeros((NROW, W), _f32).at[:N].set(htab)
    zrow = jnp.zeros((RPT, W), _f32)

    mesh = plsc.VectorSubcoreMesh(core_axis_name="c", subcore_axis_name="s",
                                  num_cores=2, num_subcores=16)
    sc1 = functools.partial(
        pl.kernel,
        out_type=jax.ShapeDtypeStruct((2, NROW, W), _f32),
        mesh=mesh,
        scratch_types=[
            pltpu.VMEM_SHARED((NROW, W), _f32),
            pltpu.VMEM((CH,), _i32),
            pltpu.VMEM((CH,), _i32),
            pltpu.VMEM((CH,), _f32),
            pltpu.VMEM((CH, W), _f32),
            pltpu.VMEM((CH, W), _f32),
            pltpu.VMEM((CH, W), _f32),
            pltpu.VMEM((16,), _f32),
            pltpu.SemaphoreType.DMA,
        ],
    )(_sc1_body)
    acc1 = sc1(htab_p, c1v, src, dst, ew, zrow)

    hid, tab2 = pl.pallas_call(
        _tc2_body,
        grid=(G,),
        in_specs=[pl.BlockSpec((2, BLK, W), lambda i: (0, i, 0)),
                  pl.BlockSpec((64, 16), lambda i: (0, 0)),
                  pl.BlockSpec((8, 64), lambda i: (0, 0)),
                  pl.BlockSpec((1, 64), lambda i: (0, 0)),
                  pl.BlockSpec((16, 1), lambda i: (0, 0)),
                  pl.BlockSpec((16, 1), lambda i: (0, 0))],
        out_specs=[pl.BlockSpec((BLK, 64), lambda i: (i, 0)),
                   pl.BlockSpec((BLK, W), lambda i: (i, 0))],
        out_shape=[jax.ShapeDtypeStruct((N, 64), _f32),
                   jax.ShapeDtypeStruct((N, W), _f32)],
    )(acc1, W2, R, b1.reshape(1, 64), a_src2.reshape(16, 1),
      a_dst2.reshape(16, 1))

    mx2v = jnp.max(tab2[:, 16:18], axis=0)
    c2 = _lrelu(mx2v[0] + mx2v[1])
    c2v = jnp.concatenate([c2.reshape(1), jnp.zeros((15,), _f32)])
    tab2_p = jnp.zeros((NROW, W), _f32).at[:N].set(tab2)

    sc2 = functools.partial(
        pl.kernel,
        out_type=jax.ShapeDtypeStruct((2, NROW, W), _f32),
        mesh=mesh,
        scratch_types=[
            pltpu.VMEM_SHARED((NROW, W), _f32),
            pltpu.VMEM((CH,), _i32),
            pltpu.VMEM((CH,), _i32),
            pltpu.VMEM((CH,), _f32),
            pltpu.VMEM((CH, W), _f32),
            pltpu.VMEM((CH, W), _f32),
            pltpu.VMEM((CH, W), _f32),
            pltpu.VMEM((16,), _f32),
            pltpu.SemaphoreType.DMA,
        ],
    )(_sc2_body)
    acc2 = sc2(tab2_p, c2v, src, dst, ew, zrow)

    b2p = jnp.zeros((1, W), _f32).at[0, :16].set(b2)
    out128 = pl.pallas_call(
        _tc3_body,
        grid=(G,),
        in_specs=[pl.BlockSpec((2, BLK, W), lambda i: (0, i, 0)),
                  pl.BlockSpec((1, W), lambda i: (0, 0))],
        out_specs=pl.BlockSpec((BLK, W), lambda i: (i, 0)),
        out_shape=jax.ShapeDtypeStruct((N, W), _f32),
    )(acc2, b2p)

    return (out128[:, :16], hid)


# speedup vs baseline: 31.1194x; 1.1295x over previous
import functools

import jax
import jax.numpy as jnp
from jax import lax
from jax.experimental import pallas as pl
from jax.experimental.pallas import tpu as pltpu
from jax.experimental.pallas import tpu_sc as plsc

N = 10000
E = 160000
F_IN = 256
HEADS = 8
HID = 8
NCLS = 16
NEG = 0.2

CH = 64
NT = 32
E1 = E + N
K1 = -(-E1 // (NT * CH))
E_PAD = NT * CH * K1
NROW = 10112
RPT = NROW // 16
W = 128

_f32 = jnp.float32
_i32 = jnp.int32


def _lrelu(a):
    return jnp.where(a > 0, a, NEG * a)


def _take(x, idx):
    return x.at[idx].get(mode="promise_in_bounds")


def _tc1_body(x_ref, w1_ref, as_ref, ad_ref, htab_ref):
    h = jnp.dot(x_ref[...], w1_ref[...], preferred_element_type=_f32)
    es = jnp.dot(h, as_ref[...], preferred_element_type=_f32)
    ed = jnp.dot(h, ad_ref[...], preferred_element_type=_f32)
    e = jnp.concatenate([es, ed], axis=1)
    z48 = jnp.zeros_like(h[:, :48])
    htab_ref[...] = jnp.concatenate([h, e, z48], axis=1)


def _tc2_body(acc_ref, w2_ref, r_ref, b1_ref, a2s_ref, a2d_ref,
              hid_ref, t2_ref):
    s = acc_ref[0] + acc_ref[1]
    msg = s[:, :64]
    den = s[:, 64:72]
    deni = 1.0 / (den + 1e-16)
    rep = jnp.dot(deni, r_ref[...], preferred_element_type=_f32)
    v = msg * rep + b1_ref[...]
    hid = jnp.where(v > 0, v, jnp.exp(v) - 1.0)
    hid_ref[...] = hid
    h2 = jnp.dot(hid, w2_ref[...], preferred_element_type=_f32)
    e2s = jnp.dot(h2, a2s_ref[...], preferred_element_type=_f32)
    e2d = jnp.dot(h2, a2d_ref[...], preferred_element_type=_f32)
    z110 = jnp.zeros((h2.shape[0], 110), _f32)
    t2_ref[...] = jnp.concatenate([h2, e2s, e2d, z110], axis=1)


def _tc3_body(acc_ref, b2_ref, out_ref):
    s = acc_ref[0] + acc_ref[1]
    den = s[:, 16:17]
    out_ref[...] = s * (1.0 / (den + 1e-16)) + b2_ref[...]


def _sc1_body(htab_hbm, c1_hbm, src_hbm, dst_hbm, ew_hbm, z_hbm,
              out_hbm,
              acc_sh,
              srcb0, dstb0, ewb0, hbuf0, edb0,
              srcb1, dstb1, ewb1, hbuf1, edb1,
              valb, c1b, sem0, sem1):
    cidx = lax.axis_index("c")
    sidx = lax.axis_index("s")
    wid = sidx * 2 + cidx
    bufs = ((srcb0, dstb0, ewb0, hbuf0, edb0, sem0),
            (srcb1, dstb1, ewb1, hbuf1, edb1, sem1))

    pltpu.sync_copy(c1_hbm, c1b)
    pltpu.sync_copy(z_hbm, acc_sh.at[pl.ds(sidx * RPT, RPT)])
    pltpu.sync_copy(z_hbm.at[pl.ds(0, CH)], valb)
    plsc.subcore_barrier()

    base = wid * (K1 * CH)
    iota = lax.iota(_i32, 16)
    c1vec = c1b[...]
    shf = (iota & 7) + 8
    widx = [(iota >> 3) + 2 * v for v in range(4)]

    def start(k, b):
        srcb, dstb, ewb, hbuf, edb, sem = bufs[b]
        off = base + k * CH
        pltpu.sync_copy(src_hbm.at[pl.ds(off, CH)], srcb)
        pltpu.sync_copy(dst_hbm.at[pl.ds(off, CH)], dstb)
        pltpu.sync_copy(ew_hbm.at[pl.ds(off, CH)], ewb)
        pltpu.async_copy(htab_hbm.at[srcb], hbuf, sem)
        pltpu.async_copy(htab_hbm.at[dstb], edb, sem)

    start(0, 0)
    start(1, 1)

    def pair(k2, carry):
      for b in range(2):
        srcb, dstb, ewb, hbuf, edb, sem = bufs[b]
        pltpu.make_async_copy(htab_hbm.at[srcb], hbuf, sem).wait()
        pltpu.make_async_copy(htab_hbm.at[dstb], edb, sem).wait()

        def group(g, c0):
            ew16 = ewb[pl.ds(g * 16, 16)]

            def edge(j, c1):
                e = g * 16 + j
                esr = hbuf[e, pl.ds(64, 16)]
                edr = edb[e, pl.ds(64, 16)]
                s = esr + _take(edr, shf)
                av = _lrelu(s) - c1vec
                ex = jnp.exp(av)
                wj = _take(ew16, jnp.zeros((16,), _i32) + j)
                exw = ex * wj
                for v in range(4):
                    hv = hbuf[e, pl.ds(v * 16, 16)]
                    valb[e, pl.ds(v * 16, 16)] = hv * _take(exw, widx[v])
                valb[e, pl.ds(64, 16)] = ex
                return c1

            lax.fori_loop(0, 16, edge, 0)
            return c0

        lax.fori_loop(0, CH // 16, group, 0)
        pltpu.sync_copy(valb, acc_sh.at[dstb], add=True)

        @pl.when(k2 * 2 + b + 2 < K1)
        def _(b=b):
            start(k2 * 2 + b + 2, b)
      return carry

    lax.fori_loop(0, K1 // 2, pair, 0)
    plsc.subcore_barrier()
    pltpu.sync_copy(acc_sh.at[pl.ds(sidx * RPT, RPT)],
                    out_hbm.at[cidx, pl.ds(sidx * RPT, RPT)])


def _sc2_body(t2_hbm, c2_hbm, src_hbm, dst_hbm, ew_hbm, z_hbm,
              out_hbm,
              acc_sh, srcb, dstb, ewb, abuf, bbuf, valb, c2b, sem):
    cidx = lax.axis_index("c")
    sidx = lax.axis_index("s")
    wid = sidx * 2 + cidx

    pltpu.sync_copy(c2_hbm, c2b)
    pltpu.sync_copy(z_hbm, acc_sh.at[pl.ds(sidx * RPT, RPT)])
    pltpu.sync_copy(z_hbm.at[pl.ds(0, CH)], valb)
    plsc.subcore_barrier()

    base = wid * (K1 * CH)
    iota = lax.iota(_i32, 16)
    zidx = iota & 0
    onei = zidx + 1
    c2vec = c2b[...]

    def chunk(k, carry):
        off = base + k * CH
        pltpu.sync_copy(src_hbm.at[pl.ds(off, CH)], srcb)
        pltpu.sync_copy(dst_hbm.at[pl.ds(off, CH)], dstb)
        pltpu.sync_copy(ew_hbm.at[pl.ds(off, CH)], ewb)
        d1 = pltpu.async_copy(t2_hbm.at[srcb], abuf, sem)
        d2 = pltpu.async_copy(t2_hbm.at[dstb], bbuf, sem)
        d1.wait()
        d2.wait()
        def group(g, c0):
            ew16 = ewb[pl.ds(g * 16, 16)]

            def edge(j, c1):
                e = g * 16 + j
                m = abuf[e, pl.ds(0, 16)]
                sv = abuf[e, pl.ds(16, 16)]
                edr = bbuf[e, pl.ds(16, 16)]
                s = sv + _take(edr, onei)
                av = _lrelu(s) - c2vec
                ex = jnp.exp(av)
                wj = _take(ew16, jnp.zeros((16,), _i32) + j)
                exw = ex * wj
                valb[e, pl.ds(0, 16)] = m * _take(exw, zidx)
                valb[e, pl.ds(16, 16)] = ex
                return c1

            lax.fori_loop(0, 16, edge, 0)
            return c0

        lax.fori_loop(0, CH // 16, group, 0)
        pltpu.sync_copy(valb, acc_sh.at[dstb], add=True)
        return carry

    lax.fori_loop(0, K1, chunk, 0)
    plsc.subcore_barrier()
    pltpu.sync_copy(acc_sh.at[pl.ds(sidx * RPT, RPT)],
                    out_hbm.at[cidx, pl.ds(sidx * RPT, RPT)])


def kernel(x, edge_index, edge_weight, W1, a_src1, a_dst1, b1,
           W2, a_src2, a_dst2, b2):
    loop = jnp.arange(N, dtype=edge_index.dtype)
    pad = E_PAD - E1
    src = jnp.concatenate([edge_index[0], loop,
                           jnp.zeros((pad,), edge_index.dtype)])
    dst = jnp.concatenate([edge_index[1], loop,
                           jnp.full((pad,), N, edge_index.dtype)])
    ew = jnp.concatenate([edge_weight, jnp.ones((N,), _f32),
                          jnp.zeros((pad,), _f32)])

    eye8 = jnp.eye(HEADS, dtype=_f32)
    A_s = (a_src1[:, :, None] * eye8[:, None, :]).reshape(HEADS * HID, HEADS)
    A_d = (a_dst1[:, :, None] * eye8[:, None, :]).reshape(HEADS * HID, HEADS)
    R = jnp.repeat(eye8, HID, axis=1)

    BLK = 1000
    G = N // BLK

    htab = pl.pallas_call(
        _tc1_body,
        grid=(G,),
        in_specs=[pl.BlockSpec((BLK, F_IN), lambda i: (i, 0)),
                  pl.BlockSpec((F_IN, 64), lambda i: (0, 0)),
                  pl.BlockSpec((64, 8), lambda i: (0, 0)),
                  pl.BlockSpec((64, 8), lambda i: (0, 0))],
        out_specs=pl.BlockSpec((BLK, W), lambda i: (i, 0)),
        out_shape=jax.ShapeDtypeStruct((N, W), _f32),
    )(x, W1, A_s, A_d)

    mxv = jnp.max(htab[:, 64:80], axis=0)
    c1 = _lrelu(mxv[:8] + mxv[8:])
    c1v = jnp.concatenate([c1, jnp.zeros((8,), _f32)])
    htab_p = jnp.zeros((NROW, W), _f32).at[:N].set(htab)
    zrow = jnp.zeros((RPT, W), _f32)

    mesh = plsc.VectorSubcoreMesh(core_axis_name="c", subcore_axis_name="s",
                                  num_cores=2, num_subcores=16)
    sc1 = functools.partial(
        pl.kernel,
        out_type=jax.ShapeDtypeStruct((2, NROW, W), _f32),
        mesh=mesh,
        scratch_types=[
            pltpu.VMEM_SHARED((NROW, W), _f32),
            pltpu.VMEM((CH,), _i32),
            pltpu.VMEM((CH,), _i32),
            pltpu.VMEM((CH,), _f32),
            pltpu.VMEM((CH, W), _f32),
            pltpu.VMEM((CH, W), _f32),
            pltpu.VMEM((CH,), _i32),
            pltpu.VMEM((CH,), _i32),
            pltpu.VMEM((CH,), _f32),
            pltpu.VMEM((CH, W), _f32),
            pltpu.VMEM((CH, W), _f32),
            pltpu.VMEM((CH, W), _f32),
            pltpu.VMEM((16,), _f32),
            pltpu.SemaphoreType.DMA,
            pltpu.SemaphoreType.DMA,
        ],
    )(_sc1_body)
    acc1 = sc1(htab_p, c1v, src, dst, ew, zrow)

    hid, tab2 = pl.pallas_call(
        _tc2_body,
        grid=(G,),
        in_specs=[pl.BlockSpec((2, BLK, W), lambda i: (0, i, 0)),
                  pl.BlockSpec((64, 16), lambda i: (0, 0)),
                  pl.BlockSpec((8, 64), lambda i: (0, 0)),
                  pl.BlockSpec((1, 64), lambda i: (0, 0)),
                  pl.BlockSpec((16, 1), lambda i: (0, 0)),
                  pl.BlockSpec((16, 1), lambda i: (0, 0))],
        out_specs=[pl.BlockSpec((BLK, 64), lambda i: (i, 0)),
                   pl.BlockSpec((BLK, W), lambda i: (i, 0))],
        out_shape=[jax.ShapeDtypeStruct((N, 64), _f32),
                   jax.ShapeDtypeStruct((N, W), _f32)],
    )(acc1, W2, R, b1.reshape(1, 64), a_src2.reshape(16, 1),
      a_dst2.reshape(16, 1))

    mx2v = jnp.max(tab2[:, 16:18], axis=0)
    c2 = _lrelu(mx2v[0] + mx2v[1])
    c2v = jnp.concatenate([c2.reshape(1), jnp.zeros((15,), _f32)])
    tab2_p = jnp.zeros((NROW, W), _f32).at[:N].set(tab2)

    sc2 = functools.partial(
        pl.kernel,
        out_type=jax.ShapeDtypeStruct((2, NROW, W), _f32),
        mesh=mesh,
        scratch_types=[
            pltpu.VMEM_SHARED((NROW, W), _f32),
            pltpu.VMEM((CH,), _i32),
            pltpu.VMEM((CH,), _i32),
            pltpu.VMEM((CH,), _f32),
            pltpu.VMEM((CH, W), _f32),
            pltpu.VMEM((CH, W), _f32),
            pltpu.VMEM((CH, W), _f32),
            pltpu.VMEM((16,), _f32),
            pltpu.SemaphoreType.DMA,
        ],
    )(_sc2_body)
    acc2 = sc2(tab2_p, c2v, src, dst, ew, zrow)

    b2p = jnp.zeros((1, W), _f32).at[0, :16].set(b2)
    out128 = pl.pallas_call(
        _tc3_body,
        grid=(G,),
        in_specs=[pl.BlockSpec((2, BLK, W), lambda i: (0, i, 0)),
                  pl.BlockSpec((1, W), lambda i: (0, 0))],
        out_specs=pl.BlockSpec((BLK, W), lambda i: (i, 0)),
        out_shape=jax.ShapeDtypeStruct((N, W), _f32),
    )(acc2, b2p)

    return (out128[:, :16], hid)


# speedup vs baseline: 38.2343x; 1.2286x over previous
import functools

import jax
import jax.numpy as jnp
from jax import lax
from jax.experimental import pallas as pl
from jax.experimental.pallas import tpu as pltpu
from jax.experimental.pallas import tpu_sc as plsc

N = 10000
E = 160000
F_IN = 256
HEADS = 8
HID = 8
NCLS = 16
NEG = 0.2

CH = 64
NT = 32
E1 = E + N
K1 = -(-E1 // (NT * CH))
E_PAD = NT * CH * K1
NROW = 10112
RPT = NROW // 16
W = 128

_f32 = jnp.float32
_i32 = jnp.int32


def _lrelu(a):
    return jnp.where(a > 0, a, NEG * a)


def _take(x, idx):
    return x.at[idx].get(mode="promise_in_bounds")


def _tc1_body(x_ref, w1_ref, as_ref, ad_ref, htab_ref):
    h = jnp.dot(x_ref[...], w1_ref[...], preferred_element_type=_f32)
    es = jnp.dot(h, as_ref[...], preferred_element_type=_f32)
    ed = jnp.dot(h, ad_ref[...], preferred_element_type=_f32)
    e = jnp.concatenate([es, ed], axis=1)
    z48 = jnp.zeros_like(h[:, :48])
    htab_ref[...] = jnp.concatenate([h, e, z48], axis=1)


def _tc2_body(acc_ref, w2_ref, r_ref, b1_ref, a2s_ref, a2d_ref,
              hid_ref, t2_ref):
    s = acc_ref[0] + acc_ref[1]
    msg = s[:, :64]
    den = s[:, 64:72]
    deni = 1.0 / (den + 1e-16)
    rep = jnp.dot(deni, r_ref[...], preferred_element_type=_f32)
    v = msg * rep + b1_ref[...]
    hid = jnp.where(v > 0, v, jnp.exp(v) - 1.0)
    hid_ref[...] = hid
    h2 = jnp.dot(hid, w2_ref[...], preferred_element_type=_f32)
    e2s = jnp.dot(h2, a2s_ref[...], preferred_element_type=_f32)
    e2d = jnp.dot(h2, a2d_ref[...], preferred_element_type=_f32)
    z110 = jnp.zeros((h2.shape[0], 110), _f32)
    t2_ref[...] = jnp.concatenate([h2, e2s, e2d, z110], axis=1)


def _tc3_body(acc_ref, b2_ref, out_ref):
    s = acc_ref[0] + acc_ref[1]
    den = s[:, 16:17]
    out_ref[...] = s * (1.0 / (den + 1e-16)) + b2_ref[...]


def _sc1_body(htab_hbm, c1_hbm, src_hbm, dst_hbm, ew_hbm, z_hbm,
              out_hbm,
              acc_sh,
              srcb0, dstb0, ewb0, hbuf0, edb0,
              srcb1, dstb1, ewb1, hbuf1, edb1,
              valb, c1b, sem0, sem1):
    cidx = lax.axis_index("c")
    sidx = lax.axis_index("s")
    wid = sidx * 2 + cidx
    bufs = ((srcb0, dstb0, ewb0, hbuf0, edb0, sem0),
            (srcb1, dstb1, ewb1, hbuf1, edb1, sem1))

    pltpu.sync_copy(c1_hbm, c1b)
    pltpu.sync_copy(z_hbm, acc_sh.at[pl.ds(sidx * RPT, RPT)])
    pltpu.sync_copy(z_hbm.at[pl.ds(0, CH)], valb)
    plsc.subcore_barrier()

    base = wid * (K1 * CH)
    iota = lax.iota(_i32, 16)
    c1vec = c1b[...]
    shf = (iota & 7) + 8
    widx = [(iota >> 3) + 2 * v for v in range(4)]

    def start(k, b):
        srcb, dstb, ewb, hbuf, edb, sem = bufs[b]
        off = base + k * CH
        pltpu.sync_copy(src_hbm.at[pl.ds(off, CH)], srcb)
        pltpu.sync_copy(dst_hbm.at[pl.ds(off, CH)], dstb)
        pltpu.sync_copy(ew_hbm.at[pl.ds(off, CH)], ewb)
        pltpu.async_copy(htab_hbm.at[srcb], hbuf, sem)
        pltpu.async_copy(htab_hbm.at[dstb], edb, sem)

    start(0, 0)
    start(1, 1)

    def pair(k2, carry):
      for b in range(2):
        srcb, dstb, ewb, hbuf, edb, sem = bufs[b]
        pltpu.make_async_copy(htab_hbm.at[srcb], hbuf, sem).wait()
        pltpu.make_async_copy(htab_hbm.at[dstb], edb, sem).wait()

        def group(g, c0):
            ew16 = ewb[pl.ds(g * 16, 16)]

            def edge(j, c1):
                e = g * 16 + j
                esr = hbuf[e, pl.ds(64, 16)]
                edr = edb[e, pl.ds(64, 16)]
                s = esr + _take(edr, shf)
                av = _lrelu(s) - c1vec
                ex = jnp.exp(av)
                wj = _take(ew16, jnp.zeros((16,), _i32) + j)
                exw = ex * wj
                for v in range(4):
                    hv = hbuf[e, pl.ds(v * 16, 16)]
                    valb[e, pl.ds(v * 16, 16)] = hv * _take(exw, widx[v])
                valb[e, pl.ds(64, 16)] = ex
                return c1

            lax.fori_loop(0, 16, edge, 0)
            return c0

        lax.fori_loop(0, CH // 16, group, 0)
        pltpu.sync_copy(valb, acc_sh.at[dstb], add=True)

        @pl.when(k2 * 2 + b + 2 < K1)
        def _(b=b):
            start(k2 * 2 + b + 2, b)
      return carry

    lax.fori_loop(0, K1 // 2, pair, 0)
    plsc.subcore_barrier()
    pltpu.sync_copy(acc_sh.at[pl.ds(sidx * RPT, RPT)],
                    out_hbm.at[cidx, pl.ds(sidx * RPT, RPT)])


def _sc2_body(t2_hbm, c2_hbm, src_hbm, dst_hbm, ew_hbm, z_hbm,
              out_hbm,
              acc_sh,
              srcb0, dstb0, ewb0, abuf0, bbuf0,
              srcb1, dstb1, ewb1, abuf1, bbuf1,
              valb, c2b, sem0, sem1):
    cidx = lax.axis_index("c")
    sidx = lax.axis_index("s")
    wid = sidx * 2 + cidx
    bufs = ((srcb0, dstb0, ewb0, abuf0, bbuf0, sem0),
            (srcb1, dstb1, ewb1, abuf1, bbuf1, sem1))

    pltpu.sync_copy(c2_hbm, c2b)
    pltpu.sync_copy(z_hbm, acc_sh.at[pl.ds(sidx * RPT, RPT)])
    pltpu.sync_copy(z_hbm.at[pl.ds(0, CH)], valb)
    plsc.subcore_barrier()

    base = wid * (K1 * CH)
    iota = lax.iota(_i32, 16)
    zidx = iota & 0
    onei = zidx + 1
    c2vec = c2b[...]

    def start(k, b):
        srcb, dstb, ewb, abuf, bbuf, sem = bufs[b]
        off = base + k * CH
        pltpu.sync_copy(src_hbm.at[pl.ds(off, CH)], srcb)
        pltpu.sync_copy(dst_hbm.at[pl.ds(off, CH)], dstb)
        pltpu.sync_copy(ew_hbm.at[pl.ds(off, CH)], ewb)
        pltpu.async_copy(t2_hbm.at[srcb], abuf, sem)
        pltpu.async_copy(t2_hbm.at[dstb], bbuf, sem)

    start(0, 0)
    start(1, 1)

    def pair(k2, carry):
      for b in range(2):
        srcb, dstb, ewb, abuf, bbuf, sem = bufs[b]
        pltpu.make_async_copy(t2_hbm.at[srcb], abuf, sem).wait()
        pltpu.make_async_copy(t2_hbm.at[dstb], bbuf, sem).wait()

        def group(g, c0):
            ew16 = ewb[pl.ds(g * 16, 16)]

            def edge(j, c1):
                e = g * 16 + j
                m = abuf[e, pl.ds(0, 16)]
                sv = abuf[e, pl.ds(16, 16)]
                edr = bbuf[e, pl.ds(16, 16)]
                s = sv + _take(edr, onei)
                av = _lrelu(s) - c2vec
                ex = jnp.exp(av)
                wj = _take(ew16, jnp.zeros((16,), _i32) + j)
                exw = ex * wj
                valb[e, pl.ds(0, 16)] = m * _take(exw, zidx)
                valb[e, pl.ds(16, 16)] = ex
                return c1

            lax.fori_loop(0, 16, edge, 0)
            return c0

        lax.fori_loop(0, CH // 16, group, 0)
        pltpu.sync_copy(valb, acc_sh.at[dstb], add=True)

        @pl.when(k2 * 2 + b + 2 < K1)
        def _(b=b):
            start(k2 * 2 + b + 2, b)
      return carry

    lax.fori_loop(0, K1 // 2, pair, 0)
    plsc.subcore_barrier()
    pltpu.sync_copy(acc_sh.at[pl.ds(sidx * RPT, RPT)],
                    out_hbm.at[cidx, pl.ds(sidx * RPT, RPT)])


def kernel(x, edge_index, edge_weight, W1, a_src1, a_dst1, b1,
           W2, a_src2, a_dst2, b2):
    loop = jnp.arange(N, dtype=edge_index.dtype)
    pad = E_PAD - E1
    src = jnp.concatenate([edge_index[0], loop,
                           jnp.zeros((pad,), edge_index.dtype)])
    dst = jnp.concatenate([edge_index[1], loop,
                           jnp.full((pad,), N, edge_index.dtype)])
    ew = jnp.concatenate([edge_weight, jnp.ones((N,), _f32),
                          jnp.zeros((pad,), _f32)])

    eye8 = jnp.eye(HEADS, dtype=_f32)
    A_s = (a_src1[:, :, None] * eye8[:, None, :]).reshape(HEADS * HID, HEADS)
    A_d = (a_dst1[:, :, None] * eye8[:, None, :]).reshape(HEADS * HID, HEADS)
    R = jnp.repeat(eye8, HID, axis=1)

    BLK = 1000
    G = N // BLK

    htab = pl.pallas_call(
        _tc1_body,
        grid=(G,),
        in_specs=[pl.BlockSpec((BLK, F_IN), lambda i: (i, 0)),
                  pl.BlockSpec((F_IN, 64), lambda i: (0, 0)),
                  pl.BlockSpec((64, 8), lambda i: (0, 0)),
                  pl.BlockSpec((64, 8), lambda i: (0, 0))],
        out_specs=pl.BlockSpec((BLK, W), lambda i: (i, 0)),
        out_shape=jax.ShapeDtypeStruct((N, W), _f32),
    )(x, W1, A_s, A_d)

    mxv = jnp.max(htab[:, 64:80], axis=0)
    c1 = _lrelu(mxv[:8] + mxv[8:])
    c1v = jnp.concatenate([c1, jnp.zeros((8,), _f32)])
    htab_p = jnp.zeros((NROW, W), _f32).at[:N].set(htab)
    zrow = jnp.zeros((RPT, W), _f32)

    mesh = plsc.VectorSubcoreMesh(core_axis_name="c", subcore_axis_name="s",
                                  num_cores=2, num_subcores=16)
    sc1 = functools.partial(
        pl.kernel,
        out_type=jax.ShapeDtypeStruct((2, NROW, W), _f32),
        mesh=mesh,
        scratch_types=[
            pltpu.VMEM_SHARED((NROW, W), _f32),
            pltpu.VMEM((CH,), _i32),
            pltpu.VMEM((CH,), _i32),
            pltpu.VMEM((CH,), _f32),
            pltpu.VMEM((CH, W), _f32),
            pltpu.VMEM((CH, W), _f32),
            pltpu.VMEM((CH,), _i32),
            pltpu.VMEM((CH,), _i32),
            pltpu.VMEM((CH,), _f32),
            pltpu.VMEM((CH, W), _f32),
            pltpu.VMEM((CH, W), _f32),
            pltpu.VMEM((CH, W), _f32),
            pltpu.VMEM((16,), _f32),
            pltpu.SemaphoreType.DMA,
            pltpu.SemaphoreType.DMA,
        ],
    )(_sc1_body)
    acc1 = sc1(htab_p, c1v, src, dst, ew, zrow)

    hid, tab2 = pl.pallas_call(
        _tc2_body,
        grid=(G,),
        in_specs=[pl.BlockSpec((2, BLK, W), lambda i: (0, i, 0)),
                  pl.BlockSpec((64, 16), lambda i: (0, 0)),
                  pl.BlockSpec((8, 64), lambda i: (0, 0)),
                  pl.BlockSpec((1, 64), lambda i: (0, 0)),
                  pl.BlockSpec((16, 1), lambda i: (0, 0)),
                  pl.BlockSpec((16, 1), lambda i: (0, 0))],
        out_specs=[pl.BlockSpec((BLK, 64), lambda i: (i, 0)),
                   pl.BlockSpec((BLK, W), lambda i: (i, 0))],
        out_shape=[jax.ShapeDtypeStruct((N, 64), _f32),
                   jax.ShapeDtypeStruct((N, W), _f32)],
    )(acc1, W2, R, b1.reshape(1, 64), a_src2.reshape(16, 1),
      a_dst2.reshape(16, 1))

    mx2v = jnp.max(tab2[:, 16:18], axis=0)
    c2 = _lrelu(mx2v[0] + mx2v[1])
    c2v = jnp.concatenate([c2.reshape(1), jnp.zeros((15,), _f32)])
    tab2_p = jnp.zeros((NROW, W), _f32).at[:N].set(tab2)

    sc2 = functools.partial(
        pl.kernel,
        out_type=jax.ShapeDtypeStruct((2, NROW, W), _f32),
        mesh=mesh,
        scratch_types=[
            pltpu.VMEM_SHARED((NROW, W), _f32),
            pltpu.VMEM((CH,), _i32),
            pltpu.VMEM((CH,), _i32),
            pltpu.VMEM((CH,), _f32),
            pltpu.VMEM((CH, W), _f32),
            pltpu.VMEM((CH, W), _f32),
            pltpu.VMEM((CH,), _i32),
            pltpu.VMEM((CH,), _i32),
            pltpu.VMEM((CH,), _f32),
            pltpu.VMEM((CH, W), _f32),
            pltpu.VMEM((CH, W), _f32),
            pltpu.VMEM((CH, W), _f32),
            pltpu.VMEM((16,), _f32),
            pltpu.SemaphoreType.DMA,
            pltpu.SemaphoreType.DMA,
        ],
    )(_sc2_body)
    acc2 = sc2(tab2_p, c2v, src, dst, ew, zrow)

    b2p = jnp.zeros((1, W), _f32).at[0, :16].set(b2)
    out128 = pl.pallas_call(
        _tc3_body,
        grid=(G,),
        in_specs=[pl.BlockSpec((2, BLK, W), lambda i: (0, i, 0)),
                  pl.BlockSpec((1, W), lambda i: (0, 0))],
        out_specs=pl.BlockSpec((BLK, W), lambda i: (i, 0)),
        out_shape=jax.ShapeDtypeStruct((N, W), _f32),
    )(acc2, b2p)

    return (out128[:, :16], hid)


# speedup vs baseline: 38.6270x; 1.0103x over previous
import functools

import jax
import jax.numpy as jnp
from jax import lax
from jax.experimental import pallas as pl
from jax.experimental.pallas import tpu as pltpu
from jax.experimental.pallas import tpu_sc as plsc

N = 10000
E = 160000
F_IN = 256
HEADS = 8
HID = 8
NCLS = 16
NEG = 0.2

CH = 64
NT = 32
E1 = E + N
K1 = -(-E1 // (NT * CH))
E_PAD = NT * CH * K1
NROW = 10112
RPT = NROW // 16
W = 128

_f32 = jnp.float32
_i32 = jnp.int32


def _lrelu(a):
    return jnp.where(a > 0, a, NEG * a)


def _take(x, idx):
    return x.at[idx].get(mode="promise_in_bounds")


def _tc1_body(x_ref, w1_ref, as_ref, ad_ref, htab_ref):
    h = jnp.dot(x_ref[...], w1_ref[...], preferred_element_type=_f32)
    es = jnp.dot(h, as_ref[...], preferred_element_type=_f32)
    ed = jnp.dot(h, ad_ref[...], preferred_element_type=_f32)
    e = jnp.concatenate([es, ed], axis=1)
    esw = jnp.concatenate([ed, es], axis=1)
    z32 = jnp.zeros_like(h[:, :32])
    htab_ref[...] = jnp.concatenate([h, e, esw, z32], axis=1)


def _tc2_body(acc_ref, w2_ref, r_ref, b1_ref, a2s_ref, a2d_ref,
              hid_ref, t2_ref):
    s = acc_ref[0] + acc_ref[1]
    msg = s[:, :64]
    den = s[:, 64:72]
    deni = 1.0 / (den + 1e-16)
    rep = jnp.dot(deni, r_ref[...], preferred_element_type=_f32)
    v = msg * rep + b1_ref[...]
    hid = jnp.where(v > 0, v, jnp.exp(v) - 1.0)
    hid_ref[...] = hid
    h2 = jnp.dot(hid, w2_ref[...], preferred_element_type=_f32)
    e2s = jnp.dot(h2, a2s_ref[...], preferred_element_type=_f32)
    e2d = jnp.dot(h2, a2d_ref[...], preferred_element_type=_f32)
    z14 = jnp.zeros((h2.shape[0], 14), _f32)
    z94 = jnp.zeros((h2.shape[0], 94), _f32)
    t2_ref[...] = jnp.concatenate(
        [h2, e2s, e2d, z14, e2d, e2s, z94], axis=1)


def _tc3_body(acc_ref, b2_ref, out_ref):
    s = acc_ref[0] + acc_ref[1]
    den = s[:, 16:17]
    out_ref[...] = s * (1.0 / (den + 1e-16)) + b2_ref[...]


def _sc1_body(htab_hbm, c1_hbm, src_hbm, dst_hbm, ew_hbm, z_hbm,
              out_hbm,
              acc_sh,
              srcb0, dstb0, ewb0, hbuf0, edb0,
              srcb1, dstb1, ewb1, hbuf1, edb1,
              valb, c1b, sem0, sem1):
    cidx = lax.axis_index("c")
    sidx = lax.axis_index("s")
    wid = sidx * 2 + cidx
    bufs = ((srcb0, dstb0, ewb0, hbuf0, edb0, sem0),
            (srcb1, dstb1, ewb1, hbuf1, edb1, sem1))

    pltpu.sync_copy(c1_hbm, c1b)
    pltpu.sync_copy(z_hbm, acc_sh.at[pl.ds(sidx * RPT, RPT)])
    pltpu.sync_copy(z_hbm.at[pl.ds(0, CH)], valb)
    plsc.subcore_barrier()

    base = wid * (K1 * CH)
    iota = lax.iota(_i32, 16)
    c1vec = c1b[...]
    widx = [(iota >> 3) + 2 * v for v in range(4)]

    def start(k, b):
        srcb, dstb, ewb, hbuf, edb, sem = bufs[b]
        off = base + k * CH
        pltpu.sync_copy(src_hbm.at[pl.ds(off, CH)], srcb)
        pltpu.sync_copy(dst_hbm.at[pl.ds(off, CH)], dstb)
        pltpu.sync_copy(ew_hbm.at[pl.ds(off, CH)], ewb)
        pltpu.async_copy(htab_hbm.at[srcb], hbuf, sem)
        pltpu.async_copy(htab_hbm.at[dstb], edb, sem)

    start(0, 0)
    start(1, 1)

    def pair(k2, carry):
      for b in range(2):
        srcb, dstb, ewb, hbuf, edb, sem = bufs[b]
        pltpu.make_async_copy(htab_hbm.at[srcb], hbuf, sem).wait()
        pltpu.make_async_copy(htab_hbm.at[dstb], edb, sem).wait()

        def group(g, c0):
            ew16 = ewb[pl.ds(g * 16, 16)]

            def edge(j, c1):
                e = g * 16 + j
                esr = hbuf[e, pl.ds(64, 16)]
                edr = edb[e, pl.ds(80, 16)]
                s = esr + edr
                av = _lrelu(s) - c1vec
                ex = jnp.exp(av)
                wj = _take(ew16, jnp.zeros((16,), _i32) + j)
                exw = ex * wj
                for v in range(4):
                    hv = hbuf[e, pl.ds(v * 16, 16)]
                    valb[e, pl.ds(v * 16, 16)] = hv * _take(exw, widx[v])
                valb[e, pl.ds(64, 16)] = ex
                return c1

            lax.fori_loop(0, 16, edge, 0)
            return c0

        lax.fori_loop(0, CH // 16, group, 0)
        pltpu.sync_copy(valb, acc_sh.at[dstb], add=True)

        @pl.when(k2 * 2 + b + 2 < K1)
        def _(b=b):
            start(k2 * 2 + b + 2, b)
      return carry

    lax.fori_loop(0, K1 // 2, pair, 0)
    plsc.subcore_barrier()
    pltpu.sync_copy(acc_sh.at[pl.ds(sidx * RPT, RPT)],
                    out_hbm.at[cidx, pl.ds(sidx * RPT, RPT)])


def _sc2_body(t2_hbm, c2_hbm, src_hbm, dst_hbm, ew_hbm, z_hbm,
              out_hbm,
              acc_sh,
              srcb0, dstb0, ewb0, abuf0, bbuf0,
              srcb1, dstb1, ewb1, abuf1, bbuf1,
              valb, c2b, sem0, sem1):
    cidx = lax.axis_index("c")
    sidx = lax.axis_index("s")
    wid = sidx * 2 + cidx
    bufs = ((srcb0, dstb0, ewb0, abuf0, bbuf0, sem0),
            (srcb1, dstb1, ewb1, abuf1, bbuf1, sem1))

    pltpu.sync_copy(c2_hbm, c2b)
    pltpu.sync_copy(z_hbm, acc_sh.at[pl.ds(sidx * RPT, RPT)])
    pltpu.sync_copy(z_hbm.at[pl.ds(0, CH)], valb)
    plsc.subcore_barrier()

    base = wid * (K1 * CH)
    iota = lax.iota(_i32, 16)
    zidx = iota & 0
    c2vec = c2b[...]

    def start(k, b):
        srcb, dstb, ewb, abuf, bbuf, sem = bufs[b]
        off = base + k * CH
        pltpu.sync_copy(src_hbm.at[pl.ds(off, CH)], srcb)
        pltpu.sync_copy(dst_hbm.at[pl.ds(off, CH)], dstb)
        pltpu.sync_copy(ew_hbm.at[pl.ds(off, CH)], ewb)
        pltpu.async_copy(t2_hbm.at[srcb], abuf, sem)
        pltpu.async_copy(t2_hbm.at[dstb], bbuf, sem)

    start(0, 0)
    start(1, 1)

    def pair(k2, carry):
      for b in range(2):
        srcb, dstb, ewb, abuf, bbuf, sem = bufs[b]
        pltpu.make_async_copy(t2_hbm.at[srcb], abuf, sem).wait()
        pltpu.make_async_copy(t2_hbm.at[dstb], bbuf, sem).wait()

        def group(g, c0):
            ew16 = ewb[pl.ds(g * 16, 16)]

            def edge(j, c1):
                e = g * 16 + j
                m = abuf[e, pl.ds(0, 16)]
                sv = abuf[e, pl.ds(16, 16)]
                edr = bbuf[e, pl.ds(32, 16)]
                s = sv + edr
                av = _lrelu(s) - c2vec
                ex = jnp.exp(av)
                wj = _take(ew16, jnp.zeros((16,), _i32) + j)
                exw = ex * wj
                valb[e, pl.ds(0, 16)] = m * _take(exw, zidx)
                valb[e, pl.ds(16, 16)] = ex
                return c1

            lax.fori_loop(0, 16, edge, 0)
            return c0

        lax.fori_loop(0, CH // 16, group, 0)
        pltpu.sync_copy(valb, acc_sh.at[dstb], add=True)

        @pl.when(k2 * 2 + b + 2 < K1)
        def _(b=b):
            start(k2 * 2 + b + 2, b)
      return carry

    lax.fori_loop(0, K1 // 2, pair, 0)
    plsc.subcore_barrier()
    pltpu.sync_copy(acc_sh.at[pl.ds(sidx * RPT, RPT)],
                    out_hbm.at[cidx, pl.ds(sidx * RPT, RPT)])


def kernel(x, edge_index, edge_weight, W1, a_src1, a_dst1, b1,
           W2, a_src2, a_dst2, b2):
    loop = jnp.arange(N, dtype=edge_index.dtype)
    pad = E_PAD - E1
    src = jnp.concatenate([edge_index[0], loop,
                           jnp.zeros((pad,), edge_index.dtype)])
    dst = jnp.concatenate([edge_index[1], loop,
                           jnp.full((pad,), N, edge_index.dtype)])
    ew = jnp.concatenate([edge_weight, jnp.ones((N,), _f32),
                          jnp.zeros((pad,), _f32)])

    eye8 = jnp.eye(HEADS, dtype=_f32)
    A_s = (a_src1[:, :, None] * eye8[:, None, :]).reshape(HEADS * HID, HEADS)
    A_d = (a_dst1[:, :, None] * eye8[:, None, :]).reshape(HEADS * HID, HEADS)
    R = jnp.repeat(eye8, HID, axis=1)

    BLK = 1000
    G = N // BLK

    htab = pl.pallas_call(
        _tc1_body,
        grid=(G,),
        in_specs=[pl.BlockSpec((BLK, F_IN), lambda i: (i, 0)),
                  pl.BlockSpec((F_IN, 64), lambda i: (0, 0)),
                  pl.BlockSpec((64, 8), lambda i: (0, 0)),
                  pl.BlockSpec((64, 8), lambda i: (0, 0))],
        out_specs=pl.BlockSpec((BLK, W), lambda i: (i, 0)),
        out_shape=jax.ShapeDtypeStruct((N, W), _f32),
    )(x, W1, A_s, A_d)

    mxv = jnp.max(htab[:, 64:80], axis=0)
    c1 = _lrelu(mxv[:8] + mxv[8:])
    c1v = jnp.concatenate([c1, jnp.zeros((8,), _f32)])
    htab_p = jnp.zeros((NROW, W), _f32).at[:N].set(htab)
    zrow = jnp.zeros((RPT, W), _f32)

    mesh = plsc.VectorSubcoreMesh(core_axis_name="c", subcore_axis_name="s",
                                  num_cores=2, num_subcores=16)
    sc1 = functools.partial(
        pl.kernel,
        out_type=jax.ShapeDtypeStruct((2, NROW, W), _f32),
        mesh=mesh,
        scratch_types=[
            pltpu.VMEM_SHARED((NROW, W), _f32),
            pltpu.VMEM((CH,), _i32),
            pltpu.VMEM((CH,), _i32),
            pltpu.VMEM((CH,), _f32),
            pltpu.VMEM((CH, W), _f32),
            pltpu.VMEM((CH, W), _f32),
            pltpu.VMEM((CH,), _i32),
            pltpu.VMEM((CH,), _i32),
            pltpu.VMEM((CH,), _f32),
            pltpu.VMEM((CH, W), _f32),
            pltpu.VMEM((CH, W), _f32),
            pltpu.VMEM((CH, W), _f32),
            pltpu.VMEM((16,), _f32),
            pltpu.SemaphoreType.DMA,
            pltpu.SemaphoreType.DMA,
        ],
    )(_sc1_body)
    acc1 = sc1(htab_p, c1v, src, dst, ew, zrow)

    hid, tab2 = pl.pallas_call(
        _tc2_body,
        grid=(G,),
        in_specs=[pl.BlockSpec((2, BLK, W), lambda i: (0, i, 0)),
                  pl.BlockSpec((64, 16), lambda i: (0, 0)),
                  pl.BlockSpec((8, 64), lambda i: (0, 0)),
                  pl.BlockSpec((1, 64), lambda i: (0, 0)),
                  pl.BlockSpec((16, 1), lambda i: (0, 0)),
                  pl.BlockSpec((16, 1), lambda i: (0, 0))],
        out_specs=[pl.BlockSpec((BLK, 64), lambda i: (i, 0)),
                   pl.BlockSpec((BLK, W), lambda i: (i, 0))],
        out_shape=[jax.ShapeDtypeStruct((N, 64), _f32),
                   jax.ShapeDtypeStruct((N, W), _f32)],
    )(acc1, W2, R, b1.reshape(1, 64), a_src2.reshape(16, 1),
      a_dst2.reshape(16, 1))

    mx2v = jnp.max(tab2[:, 16:18], axis=0)
    c2 = _lrelu(mx2v[0] + mx2v[1])
    c2v = jnp.concatenate([c2.reshape(1), jnp.zeros((15,), _f32)])
    tab2_p = jnp.zeros((NROW, W), _f32).at[:N].set(tab2)

    sc2 = functools.partial(
        pl.kernel,
        out_type=jax.ShapeDtypeStruct((2, NROW, W), _f32),
        mesh=mesh,
        scratch_types=[
            pltpu.VMEM_SHARED((NROW, W), _f32),
            pltpu.VMEM((CH,), _i32),
            pltpu.VMEM((CH,), _i32),
            pltpu.VMEM((CH,), _f32),
            pltpu.VMEM((CH, W), _f32),
            pltpu.VMEM((CH, W), _f32),
            pltpu.VMEM((CH,), _i32),
            pltpu.VMEM((CH,), _i32),
            pltpu.VMEM((CH,), _f32),
            pltpu.VMEM((CH, W), _f32),
            pltpu.VMEM((CH, W), _f32),
            pltpu.VMEM((CH, W), _f32),
            pltpu.VMEM((16,), _f32),
            pltpu.SemaphoreType.DMA,
            pltpu.SemaphoreType.DMA,
        ],
    )(_sc2_body)
    acc2 = sc2(tab2_p, c2v, src, dst, ew, zrow)

    b2p = jnp.zeros((1, W), _f32).at[0, :16].set(b2)
    out128 = pl.pallas_call(
        _tc3_body,
        grid=(G,),
        in_specs=[pl.BlockSpec((2, BLK, W), lambda i: (0, i, 0)),
                  pl.BlockSpec((1, W), lambda i: (0, 0))],
        out_specs=pl.BlockSpec((BLK, W), lambda i: (i, 0)),
        out_shape=jax.ShapeDtypeStruct((N, W), _f32),
    )(acc2, b2p)

    return (out128[:, :16], hid)


# speedup vs baseline: 48.4439x; 1.2541x over previous
import functools

import jax
import jax.numpy as jnp
from jax import lax
from jax.experimental import pallas as pl
from jax.experimental.pallas import tpu as pltpu
from jax.experimental.pallas import tpu_sc as plsc

N = 10000
E = 160000
F_IN = 256
HEADS = 8
HID = 8
NCLS = 16
NEG = 0.2

CH = 64
NT = 32
E1 = E + N
K1 = -(-E1 // (NT * CH))
E_PAD = NT * CH * K1
NROW = 10112
RPT = NROW // 16
W = 128

_f32 = jnp.float32
_i32 = jnp.int32


def _lrelu(a):
    return jnp.where(a > 0, a, NEG * a)


def _take(x, idx):
    return x.at[idx].get(mode="promise_in_bounds")


def _tc1_body(x_ref, w1_ref, as_ref, ad_ref, htab_ref):
    h = jnp.dot(x_ref[...], w1_ref[...], preferred_element_type=_f32)
    es = jnp.dot(h, as_ref[...], preferred_element_type=_f32)
    ed = jnp.dot(h, ad_ref[...], preferred_element_type=_f32)
    e = jnp.concatenate([es, ed], axis=1)
    esw = jnp.concatenate([ed, es], axis=1)
    z32 = jnp.zeros_like(h[:, :32])
    htab_ref[...] = jnp.concatenate([h, e, esw, z32], axis=1)


def _tc2_body(acc_ref, w2_ref, r_ref, b1_ref, a2s_ref, a2d_ref,
              hid_ref, t2_ref):
    s = acc_ref[0] + acc_ref[1]
    msg = s[:, :64]
    den = s[:, 64:72]
    deni = 1.0 / (den + 1e-16)
    rep = jnp.dot(deni, r_ref[...], preferred_element_type=_f32)
    v = msg * rep + b1_ref[...]
    hid = jnp.where(v > 0, v, jnp.exp(v) - 1.0)
    hid_ref[...] = hid
    h2 = jnp.dot(hid, w2_ref[...], preferred_element_type=_f32)
    e2s = jnp.dot(h2, a2s_ref[...], preferred_element_type=_f32)
    e2d = jnp.dot(h2, a2d_ref[...], preferred_element_type=_f32)
    z14 = jnp.zeros((h2.shape[0], 14), _f32)
    z94 = jnp.zeros((h2.shape[0], 94), _f32)
    t2_ref[...] = jnp.concatenate(
        [h2, e2s, e2d, z14, e2d, e2s, z94], axis=1)


def _tc3_body(acc_ref, b2_ref, out_ref):
    s = acc_ref[0] + acc_ref[1]
    den = s[:, 16:17]
    out_ref[...] = s * (1.0 / (den + 1e-16)) + b2_ref[...]


def _sc1_body(htab_hbm, c1_hbm, src_hbm, dst_hbm, ew_hbm, z_hbm,
              out_hbm,
              acc_sh,
              srcb0, dstb0, ewb0, hbuf0, edb0,
              srcb1, dstb1, ewb1, hbuf1, edb1,
              valb, c1b, sem0, sem1):
    cidx = lax.axis_index("c")
    sidx = lax.axis_index("s")
    wid = sidx * 2 + cidx
    bufs = ((srcb0, dstb0, ewb0, hbuf0, edb0, sem0),
            (srcb1, dstb1, ewb1, hbuf1, edb1, sem1))

    pltpu.sync_copy(c1_hbm, c1b)
    pltpu.sync_copy(z_hbm, acc_sh.at[pl.ds(sidx * RPT, RPT)])
    pltpu.sync_copy(z_hbm.at[pl.ds(0, CH)], valb)
    plsc.subcore_barrier()

    base = wid * (K1 * CH)
    iota = lax.iota(_i32, 16)
    c1vec = c1b[...]
    widx = [(iota >> 3) + 2 * v for v in range(4)]

    def start(k, b):
        srcb, dstb, ewb, hbuf, edb, sem = bufs[b]
        off = base + k * CH
        pltpu.sync_copy(src_hbm.at[pl.ds(off, CH)], srcb)
        pltpu.sync_copy(dst_hbm.at[pl.ds(off, CH)], dstb)
        pltpu.sync_copy(ew_hbm.at[pl.ds(off, CH)], ewb)
        pltpu.async_copy(htab_hbm.at[srcb], hbuf, sem)
        pltpu.async_copy(htab_hbm.at[dstb], edb, sem)

    start(0, 0)
    start(1, 1)

    def pair(k2, carry):
      for b in range(2):
        srcb, dstb, ewb, hbuf, edb, sem = bufs[b]
        pltpu.make_async_copy(htab_hbm.at[srcb], hbuf, sem).wait()
        pltpu.make_async_copy(htab_hbm.at[dstb], edb, sem).wait()

        def group(g, c0):
            ew16 = ewb[pl.ds(g * 16, 16)]

            for j in range(16):
                e = g * 16 + j
                esr = hbuf[e, pl.ds(64, 16)]
                edr = edb[e, pl.ds(80, 16)]
                s = esr + edr
                av = _lrelu(s) - c1vec
                ex = jnp.exp(av)
                wj = _take(ew16, jnp.zeros((16,), _i32) + j)
                exw = ex * wj
                for v in range(4):
                    hv = hbuf[e, pl.ds(v * 16, 16)]
                    valb[e, pl.ds(v * 16, 16)] = hv * _take(exw, widx[v])
                valb[e, pl.ds(64, 16)] = ex
            return c0

        lax.fori_loop(0, CH // 16, group, 0)
        pltpu.sync_copy(valb, acc_sh.at[dstb], add=True)

        @pl.when(k2 * 2 + b + 2 < K1)
        def _(b=b):
            start(k2 * 2 + b + 2, b)
      return carry

    lax.fori_loop(0, K1 // 2, pair, 0)
    plsc.subcore_barrier()
    pltpu.sync_copy(acc_sh.at[pl.ds(sidx * RPT, RPT)],
                    out_hbm.at[cidx, pl.ds(sidx * RPT, RPT)])


def _sc2_body(t2_hbm, c2_hbm, src_hbm, dst_hbm, ew_hbm, z_hbm,
              out_hbm,
              acc_sh,
              srcb0, dstb0, ewb0, abuf0, bbuf0,
              srcb1, dstb1, ewb1, abuf1, bbuf1,
              valb, c2b, sem0, sem1):
    cidx = lax.axis_index("c")
    sidx = lax.axis_index("s")
    wid = sidx * 2 + cidx
    bufs = ((srcb0, dstb0, ewb0, abuf0, bbuf0, sem0),
            (srcb1, dstb1, ewb1, abuf1, bbuf1, sem1))

    pltpu.sync_copy(c2_hbm, c2b)
    pltpu.sync_copy(z_hbm, acc_sh.at[pl.ds(sidx * RPT, RPT)])
    pltpu.sync_copy(z_hbm.at[pl.ds(0, CH)], valb)
    plsc.subcore_barrier()

    base = wid * (K1 * CH)
    iota = lax.iota(_i32, 16)
    zidx = iota & 0
    c2vec = c2b[...]

    def start(k, b):
        srcb, dstb, ewb, abuf, bbuf, sem = bufs[b]
        off = base + k * CH
        pltpu.sync_copy(src_hbm.at[pl.ds(off, CH)], srcb)
        pltpu.sync_copy(dst_hbm.at[pl.ds(off, CH)], dstb)
        pltpu.sync_copy(ew_hbm.at[pl.ds(off, CH)], ewb)
        pltpu.async_copy(t2_hbm.at[srcb], abuf, sem)
        pltpu.async_copy(t2_hbm.at[dstb], bbuf, sem)

    start(0, 0)
    start(1, 1)

    def pair(k2, carry):
      for b in range(2):
        srcb, dstb, ewb, abuf, bbuf, sem = bufs[b]
        pltpu.make_async_copy(t2_hbm.at[srcb], abuf, sem).wait()
        pltpu.make_async_copy(t2_hbm.at[dstb], bbuf, sem).wait()

        def group(g, c0):
            ew16 = ewb[pl.ds(g * 16, 16)]

            for j in range(16):
                e = g * 16 + j
                m = abuf[e, pl.ds(0, 16)]
                sv = abuf[e, pl.ds(16, 16)]
                edr = bbuf[e, pl.ds(32, 16)]
                s = sv + edr
                av = _lrelu(s) - c2vec
                ex = jnp.exp(av)
                wj = _take(ew16, jnp.zeros((16,), _i32) + j)
                exw = ex * wj
                valb[e, pl.ds(0, 16)] = m * _take(exw, zidx)
                valb[e, pl.ds(16, 16)] = ex
            return c0

        lax.fori_loop(0, CH // 16, group, 0)
        pltpu.sync_copy(valb, acc_sh.at[dstb], add=True)

        @pl.when(k2 * 2 + b + 2 < K1)
        def _(b=b):
            start(k2 * 2 + b + 2, b)
      return carry

    lax.fori_loop(0, K1 // 2, pair, 0)
    plsc.subcore_barrier()
    pltpu.sync_copy(acc_sh.at[pl.ds(sidx * RPT, RPT)],
                    out_hbm.at[cidx, pl.ds(sidx * RPT, RPT)])


def kernel(x, edge_index, edge_weight, W1, a_src1, a_dst1, b1,
           W2, a_src2, a_dst2, b2):
    loop = jnp.arange(N, dtype=edge_index.dtype)
    pad = E_PAD - E1
    src = jnp.concatenate([edge_index[0], loop,
                           jnp.zeros((pad,), edge_index.dtype)])
    dst = jnp.concatenate([edge_index[1], loop,
                           jnp.full((pad,), N, edge_index.dtype)])
    ew = jnp.concatenate([edge_weight, jnp.ones((N,), _f32),
                          jnp.zeros((pad,), _f32)])

    eye8 = jnp.eye(HEADS, dtype=_f32)
    A_s = (a_src1[:, :, None] * eye8[:, None, :]).reshape(HEADS * HID, HEADS)
    A_d = (a_dst1[:, :, None] * eye8[:, None, :]).reshape(HEADS * HID, HEADS)
    R = jnp.repeat(eye8, HID, axis=1)

    BLK = 1000
    G = N // BLK

    htab = pl.pallas_call(
        _tc1_body,
        grid=(G,),
        in_specs=[pl.BlockSpec((BLK, F_IN), lambda i: (i, 0)),
                  pl.BlockSpec((F_IN, 64), lambda i: (0, 0)),
                  pl.BlockSpec((64, 8), lambda i: (0, 0)),
                  pl.BlockSpec((64, 8), lambda i: (0, 0))],
        out_specs=pl.BlockSpec((BLK, W), lambda i: (i, 0)),
        out_shape=jax.ShapeDtypeStruct((N, W), _f32),
    )(x, W1, A_s, A_d)

    mxv = jnp.max(htab[:, 64:80], axis=0)
    c1 = _lrelu(mxv[:8] + mxv[8:])
    c1v = jnp.concatenate([c1, jnp.zeros((8,), _f32)])
    htab_p = jnp.zeros((NROW, W), _f32).at[:N].set(htab)
    zrow = jnp.zeros((RPT, W), _f32)

    mesh = plsc.VectorSubcoreMesh(core_axis_name="c", subcore_axis_name="s",
                                  num_cores=2, num_subcores=16)
    sc1 = functools.partial(
        pl.kernel,
        out_type=jax.ShapeDtypeStruct((2, NROW, W), _f32),
        mesh=mesh,
        scratch_types=[
            pltpu.VMEM_SHARED((NROW, W), _f32),
            pltpu.VMEM((CH,), _i32),
            pltpu.VMEM((CH,), _i32),
            pltpu.VMEM((CH,), _f32),
            pltpu.VMEM((CH, W), _f32),
            pltpu.VMEM((CH, W), _f32),
            pltpu.VMEM((CH,), _i32),
            pltpu.VMEM((CH,), _i32),
            pltpu.VMEM((CH,), _f32),
            pltpu.VMEM((CH, W), _f32),
            pltpu.VMEM((CH, W), _f32),
            pltpu.VMEM((CH, W), _f32),
            pltpu.VMEM((16,), _f32),
            pltpu.SemaphoreType.DMA,
            pltpu.SemaphoreType.DMA,
        ],
    )(_sc1_body)
    acc1 = sc1(htab_p, c1v, src, dst, ew, zrow)

    hid, tab2 = pl.pallas_call(
        _tc2_body,
        grid=(G,),
        in_specs=[pl.BlockSpec((2, BLK, W), lambda i: (0, i, 0)),
                  pl.BlockSpec((64, 16), lambda i: (0, 0)),
                  pl.BlockSpec((8, 64), lambda i: (0, 0)),
                  pl.BlockSpec((1, 64), lambda i: (0, 0)),
                  pl.BlockSpec((16, 1), lambda i: (0, 0)),
                  pl.BlockSpec((16, 1), lambda i: (0, 0))],
        out_specs=[pl.BlockSpec((BLK, 64), lambda i: (i, 0)),
                   pl.BlockSpec((BLK, W), lambda i: (i, 0))],
        out_shape=[jax.ShapeDtypeStruct((N, 64), _f32),
                   jax.ShapeDtypeStruct((N, W), _f32)],
    )(acc1, W2, R, b1.reshape(1, 64), a_src2.reshape(16, 1),
      a_dst2.reshape(16, 1))

    mx2v = jnp.max(tab2[:, 16:18], axis=0)
    c2 = _lrelu(mx2v[0] + mx2v[1])
    c2v = jnp.concatenate([c2.reshape(1), jnp.zeros((15,), _f32)])
    tab2_p = jnp.zeros((NROW, W), _f32).at[:N].set(tab2)

    sc2 = functools.partial(
        pl.kernel,
        out_type=jax.ShapeDtypeStruct((2, NROW, W), _f32),
        mesh=mesh,
        scratch_types=[
            pltpu.VMEM_SHARED((NROW, W), _f32),
            pltpu.VMEM((CH,), _i32),
            pltpu.VMEM((CH,), _i32),
            pltpu.VMEM((CH,), _f32),
            pltpu.VMEM((CH, W), _f32),
            pltpu.VMEM((CH, W), _f32),
            pltpu.VMEM((CH,), _i32),
            pltpu.VMEM((CH,), _i32),
            pltpu.VMEM((CH,), _f32),
            pltpu.VMEM((CH, W), _f32),
            pltpu.VMEM((CH, W), _f32),
            pltpu.VMEM((CH, W), _f32),
            pltpu.VMEM((16,), _f32),
            pltpu.SemaphoreType.DMA,
            pltpu.SemaphoreType.DMA,
        ],
    )(_sc2_body)
    acc2 = sc2(tab2_p, c2v, src, dst, ew, zrow)

    b2p = jnp.zeros((1, W), _f32).at[0, :16].set(b2)
    out128 = pl.pallas_call(
        _tc3_body,
        grid=(G,),
        in_specs=[pl.BlockSpec((2, BLK, W), lambda i: (0, i, 0)),
                  pl.BlockSpec((1, W), lambda i: (0, 0))],
        out_specs=pl.BlockSpec((BLK, W), lambda i: (i, 0)),
        out_shape=jax.ShapeDtypeStruct((N, W), _f32),
    )(acc2, b2p)

    return (out128[:, :16], hid)
